# baseline (speedup 1.0000x reference)
import jax
import jax.numpy as jnp
from jax import lax
from jax.experimental import pallas as pl
from jax.experimental.pallas import tpu as pltpu
from jax.experimental.pallas import tpu_sc as plsc

NUM_ACTIONS = 6
ACTION_DIM = 32
BATCH = 16384
HIST = 200

NC = 2
NS = 16
NW = NC * NS
L = 16

G = 8
CHUNK_A = G * HIST
NBLK = CHUNK_A // L
ROW_W = 128
CHUNK_R = G * HIST * ACTION_DIM // ROW_W
NCHUNKS = BATCH // G
CPW = NCHUNKS // NW

_DNUMS = lax.GatherDimensionNumbers(
    offset_dims=(), collapsed_slice_dims=(0,), start_index_map=(0,)
)


def _vbcast(vec, lane):
    idx = jnp.full((L, 1), lane, dtype=jnp.int32)
    return lax.gather(vec, idx, _DNUMS, (1,),
                      mode=lax.GatherScatterMode.PROMISE_IN_BOUNDS)


def _sc_body(act_hbm, table_hbm, out_hbm, table_v, a0_v, a1_v, r0_v, r1_v,
             is0, is1, ws0, ws1):
    wid = lax.axis_index("s") * NC + lax.axis_index("c")
    base = wid * CPW
    i16 = lax.iota(jnp.int32, 16)
    acts, rows = (a0_v, a1_v), (r0_v, r1_v)
    isem, wsem = (is0, is1), (ws0, ws1)

    pltpu.sync_copy(table_hbm, table_v)

    def fire_idx(i, b):
        pltpu.async_copy(act_hbm.at[base + i], acts[b], isem[b])

    def drain_idx(b):
        pltpu.make_async_copy(act_hbm.at[0], acts[b], isem[b]).wait()

    def expand(b):
        act_ref, row_ref = acts[b], rows[b]

        @pl.loop(0, NBLK)
        def _blk(j):
            a_vec = act_ref[pl.ds(j * L, L)]
            for step in range(L):
                rep = _vbcast(a_vec, step)
                for h in range(2):
                    k = 2 * step + h
                    col = i16 + h * L
                    val = plsc.load_gather(table_v, [rep, col])
                    row_ref[4 * j + k // 8, pl.ds((k % 8) * L, L)] = val

    def fire_write(i, b):
        pltpu.async_copy(rows[b], out_hbm.at[base + i], wsem[b])

    def drain_write(b):
        pltpu.make_async_copy(out_hbm.at[0], rows[b], wsem[b]).wait()

    def slot(i, b, first, last):
        @pl.when(jnp.logical_not(first))
        def _():
            drain_write(b)
        drain_idx(b)
        expand(b)
        fire_write(i, b)
        @pl.when(jnp.logical_not(last))
        def _():
            fire_idx(i + 2, b)

    fire_idx(0, 0)
    fire_idx(1, 1)

    @pl.loop(0, CPW, step=2)
    def _pair(c0):
        slot(c0, 0, c0 == 0, c0 + 2 >= CPW)
        slot(c0 + 1, 1, c0 == 0, c0 + 3 >= CPW)

    drain_write(0)
    drain_write(1)


def kernel(action, table):
    act2 = action.reshape(NCHUNKS, CHUNK_A)
    kfn = pl.kernel(
        _sc_body,
        out_type=jax.ShapeDtypeStruct((NCHUNKS, CHUNK_R, ROW_W), jnp.float32),
        mesh=plsc.VectorSubcoreMesh(core_axis_name="c", subcore_axis_name="s"),
        compiler_params=pltpu.CompilerParams(needs_layout_passes=False),
        scratch_types=[
            pltpu.VMEM((NUM_ACTIONS, ACTION_DIM), jnp.float32),
            pltpu.VMEM((CHUNK_A,), jnp.int32),
            pltpu.VMEM((CHUNK_A,), jnp.int32),
            pltpu.VMEM((CHUNK_R, ROW_W), jnp.float32),
            pltpu.VMEM((CHUNK_R, ROW_W), jnp.float32),
            pltpu.SemaphoreType.DMA,
            pltpu.SemaphoreType.DMA,
            pltpu.SemaphoreType.DMA,
            pltpu.SemaphoreType.DMA,
        ],
    )
    out3 = kfn(act2, table)
    return out3.reshape(BATCH, HIST * ACTION_DIM)
